# baseline (speedup 1.0000x reference)
import functools

import jax
import jax.numpy as jnp
from jax.experimental import pallas as pl
from jax.experimental.pallas import tpu as pltpu

_EPS = 1e-5
_WOFF = 8


def _in_relu(x, n):
    axes = tuple(range(x.ndim - 1))
    s = jnp.sum(x, axis=axes, keepdims=True)
    ss = jnp.sum(x * x, axis=axes, keepdims=True)
    mu = s * (1.0 / n)
    var = ss * (1.0 / n) - mu * mu
    scale = jax.lax.rsqrt(var + _EPS)
    return jnp.maximum(x * scale - mu * scale, 0.0)


def _im2col_patches(hp_ref, H, W, Cin, stride):
    Ho, Wo = H // stride, W // stride
    patches = []
    for kh in range(3):
        for kw in range(3):
            if stride == 1:
                p = hp_ref[kh:kh + Ho, _WOFF - 1 + kw:_WOFF - 1 + kw + Wo, :]
            else:
                p = hp_ref[pl.ds(kh, Ho, stride=stride),
                           pl.ds(_WOFF - 1 + kw, Wo, stride=stride), :]
            patches.append(p.reshape(Ho * Wo, Cin).astype(jnp.bfloat16))
    return patches


def _stage1_kernel(x_ref, w1_ref, wsc_ref, out1_ref, sc_ref, hp_ref,
                   *, H, W, Cin, Cout, stride):
    Ho, Wo = H // stride, W // stride

    hp_ref[...] = jnp.zeros_like(hp_ref)
    h = _in_relu(x_ref[0], H * W)
    hp_ref[1:H + 1, _WOFF:_WOFF + W, :] = h

    patches = _im2col_patches(hp_ref, H, W, Cin, stride)
    cols = jnp.concatenate(patches, axis=-1)
    out1 = jnp.dot(cols, w1_ref[...], preferred_element_type=jnp.float32)
    sc = jnp.dot(patches[4], wsc_ref[...], preferred_element_type=jnp.float32)

    out1_ref[0] = out1.reshape(Ho, Wo, Cout).astype(out1_ref.dtype)
    sc_ref[0] = sc.astype(sc_ref.dtype)


def _stage2_kernel(x_ref, sc_ref, w2_ref, out_ref, hp_ref, *, H, W, C):
    hp_ref[...] = jnp.zeros_like(hp_ref)
    h = _in_relu(x_ref[0].astype(jnp.float32), H * W)
    hp_ref[1:H + 1, _WOFF:_WOFF + W, :] = h

    patches = _im2col_patches(hp_ref, H, W, C, 1)
    cols = jnp.concatenate(patches, axis=-1)
    out = jnp.dot(cols, w2_ref[...], preferred_element_type=jnp.float32)
    out = out + sc_ref[0].astype(jnp.float32)
    out_ref[0] = jnp.transpose(out, (1, 0)).astype(out_ref.dtype)


def kernel(x, w1, w2, w_sc):
    stride, scale = 2, 2.0
    N, Cin, H, W = x.shape
    Cout = w1.shape[0]
    Ho, Wo = H // stride, W // stride
    bf16 = jnp.bfloat16

    w1_mat = (jnp.transpose(w1, (2, 3, 1, 0)).reshape(9 * Cin, Cout) * scale).astype(bf16)
    w2_mat = (jnp.transpose(w2, (2, 3, 1, 0)).reshape(9 * Cout, Cout) * scale).astype(bf16)
    wsc_mat = (jnp.transpose(w_sc[:, :, 0, 0], (1, 0)) * scale).astype(bf16)

    x_nhwc = jnp.transpose(x, (0, 2, 3, 1))

    cparams = pltpu.CompilerParams(
        dimension_semantics=("parallel",),
        vmem_limit_bytes=32 * 1024 * 1024,
    )

    k1 = functools.partial(_stage1_kernel, H=H, W=W, Cin=Cin, Cout=Cout, stride=stride)
    out1, sc = pl.pallas_call(
        k1,
        grid=(N,),
        in_specs=[
            pl.BlockSpec((1, H, W, Cin), lambda n: (n, 0, 0, 0)),
            pl.BlockSpec((9 * Cin, Cout), lambda n: (0, 0)),
            pl.BlockSpec((Cin, Cout), lambda n: (0, 0)),
        ],
        out_specs=[
            pl.BlockSpec((1, Ho, Wo, Cout), lambda n: (n, 0, 0, 0)),
            pl.BlockSpec((1, Ho * Wo, Cout), lambda n: (n, 0, 0)),
        ],
        out_shape=[
            jax.ShapeDtypeStruct((N, Ho, Wo, Cout), bf16),
            jax.ShapeDtypeStruct((N, Ho * Wo, Cout), bf16),
        ],
        scratch_shapes=[pltpu.VMEM((H + 2, _WOFF + W + 8, Cin), jnp.float32)],
        compiler_params=cparams,
    )(x_nhwc, w1_mat, wsc_mat)

    k2 = functools.partial(_stage2_kernel, H=Ho, W=Wo, C=Cout)
    out = pl.pallas_call(
        k2,
        grid=(N,),
        in_specs=[
            pl.BlockSpec((1, Ho, Wo, Cout), lambda n: (n, 0, 0, 0)),
            pl.BlockSpec((1, Ho * Wo, Cout), lambda n: (n, 0, 0)),
            pl.BlockSpec((9 * Cout, Cout), lambda n: (0, 0)),
        ],
        out_specs=pl.BlockSpec((1, Cout, Ho * Wo), lambda n: (n, 0, 0)),
        out_shape=jax.ShapeDtypeStruct((N, Cout, Ho * Wo), x.dtype),
        scratch_shapes=[pltpu.VMEM((Ho + 2, _WOFF + Wo + 8, Cout), jnp.float32)],
        compiler_params=cparams,
    )(out1, sc, w2_mat)

    return out.reshape(N, Cout, Ho, Wo)


# speedup vs baseline: 1.2875x; 1.2875x over previous
import functools

import jax
import jax.numpy as jnp
from jax.experimental import pallas as pl
from jax.experimental.pallas import tpu as pltpu

_EPS = 1e-5
_WOFF = 8


def _in_relu(x, n):
    axes = tuple(range(x.ndim - 1))
    s = jnp.sum(x, axis=axes, keepdims=True)
    ss = jnp.sum(x * x, axis=axes, keepdims=True)
    mu = s * (1.0 / n)
    var = ss * (1.0 / n) - mu * mu
    scale = jax.lax.rsqrt(var + _EPS)
    return jnp.maximum(x * scale - mu * scale, 0.0)


def _im2col_patches(hp_ref, H, W, Cin, stride):
    Ho, Wo = H // stride, W // stride
    patches = []
    for kh in range(3):
        for kw in range(3):
            if stride == 1:
                p = hp_ref[kh:kh + Ho, _WOFF - 1 + kw:_WOFF - 1 + kw + Wo, :]
            else:
                p = hp_ref[pl.ds(kh, Ho, stride=stride),
                           pl.ds(_WOFF - 1 + kw, Wo, stride=stride), :]
            patches.append(p.reshape(Ho * Wo, Cin).astype(jnp.bfloat16))
    return patches


def _stage1_kernel(x_ref, w1_ref, wsc_ref, out1_ref, sc_ref, hp_ref,
                   *, H, W, Cin, Cout, stride):
    Ho, Wo = H // stride, W // stride

    hp_ref[...] = jnp.zeros_like(hp_ref)
    h = _in_relu(x_ref[0], H * W)
    hp_ref[1:H + 1, _WOFF:_WOFF + W, :] = h

    patches = _im2col_patches(hp_ref, H, W, Cin, stride)
    cols = jnp.concatenate(patches, axis=-1)
    out1 = jnp.dot(cols, w1_ref[...], preferred_element_type=jnp.float32)
    sc = jnp.dot(patches[4], wsc_ref[...], preferred_element_type=jnp.float32)

    out1_ref[0] = out1.reshape(Ho, Wo, Cout).astype(out1_ref.dtype)
    sc_ref[0] = sc.astype(sc_ref.dtype)


def _stage2_kernel(x_ref, sc_ref, w2_ref, out_ref, hp_ref, *, H, W, C):
    hp_ref[...] = jnp.zeros_like(hp_ref)
    h = _in_relu(x_ref[0].astype(jnp.float32), H * W)
    hp_ref[1:H + 1, _WOFF:_WOFF + W, :] = h

    patches = _im2col_patches(hp_ref, H, W, C, 1)
    cols = jnp.concatenate(patches, axis=-1)
    out = jnp.dot(cols, w2_ref[...], preferred_element_type=jnp.float32)
    out = out + sc_ref[0].astype(jnp.float32)
    out_ref[0] = out.astype(out_ref.dtype)


def kernel(x, w1, w2, w_sc):
    stride, scale = 2, 2.0
    N, Cin, H, W = x.shape
    Cout = w1.shape[0]
    Ho, Wo = H // stride, W // stride
    bf16 = jnp.bfloat16

    w1_mat = (jnp.transpose(w1, (2, 3, 1, 0)).reshape(9 * Cin, Cout) * scale).astype(bf16)
    w2_mat = (jnp.transpose(w2, (2, 3, 1, 0)).reshape(9 * Cout, Cout) * scale).astype(bf16)
    wsc_mat = (jnp.transpose(w_sc[:, :, 0, 0], (1, 0)) * scale).astype(bf16)

    x_nhwc = jnp.transpose(x, (0, 2, 3, 1))

    cparams = pltpu.CompilerParams(
        dimension_semantics=("parallel",),
        vmem_limit_bytes=32 * 1024 * 1024,
    )

    k1 = functools.partial(_stage1_kernel, H=H, W=W, Cin=Cin, Cout=Cout, stride=stride)
    out1, sc = pl.pallas_call(
        k1,
        grid=(N,),
        in_specs=[
            pl.BlockSpec((1, H, W, Cin), lambda n: (n, 0, 0, 0)),
            pl.BlockSpec((9 * Cin, Cout), lambda n: (0, 0)),
            pl.BlockSpec((Cin, Cout), lambda n: (0, 0)),
        ],
        out_specs=[
            pl.BlockSpec((1, Ho, Wo, Cout), lambda n: (n, 0, 0, 0)),
            pl.BlockSpec((1, Ho * Wo, Cout), lambda n: (n, 0, 0)),
        ],
        out_shape=[
            jax.ShapeDtypeStruct((N, Ho, Wo, Cout), bf16),
            jax.ShapeDtypeStruct((N, Ho * Wo, Cout), bf16),
        ],
        scratch_shapes=[pltpu.VMEM((H + 2, _WOFF + W + 8, Cin), jnp.float32)],
        compiler_params=cparams,
    )(x_nhwc, w1_mat, wsc_mat)

    k2 = functools.partial(_stage2_kernel, H=Ho, W=Wo, C=Cout)
    out = pl.pallas_call(
        k2,
        grid=(N,),
        in_specs=[
            pl.BlockSpec((1, Ho, Wo, Cout), lambda n: (n, 0, 0, 0)),
            pl.BlockSpec((1, Ho * Wo, Cout), lambda n: (n, 0, 0)),
            pl.BlockSpec((9 * Cout, Cout), lambda n: (0, 0)),
        ],
        out_specs=pl.BlockSpec((1, Ho * Wo, Cout), lambda n: (n, 0, 0)),
        out_shape=jax.ShapeDtypeStruct((N, Ho * Wo, Cout), x.dtype),
        scratch_shapes=[pltpu.VMEM((Ho + 2, _WOFF + Wo + 8, Cout), jnp.float32)],
        compiler_params=cparams,
    )(out1, sc, w2_mat)

    out = out.reshape(N, Ho, Wo, Cout)
    return jnp.transpose(out, (0, 3, 1, 2))
